# baseline (speedup 1.0000x reference)
import functools

import jax
import jax.numpy as jnp
from jax import lax
from jax.experimental import pallas as pl
from jax.experimental.pallas import tpu as pltpu
from jax.experimental.pallas import tpu_sc as plsc

K = 8192
WORD_NUM = 32
TEXT_DIM = 512
FRAME_NUM = 12
VID_DIM = 512
B = 4

KB = 128
NSTEPS = K // KB
ROW = FRAME_NUM * VID_DIM


def _argmin_body(q_ref, qt_ref, idx_ref, minval, lq):
    step = pl.program_id(0)

    @pl.when(step == 0)
    def _init():
        lq[...] = jnp.log(q_ref[...])
        minval[...] = jnp.full((B, TEXT_DIM), jnp.inf, jnp.float32)
        idx_ref[...] = jnp.zeros((B, TEXT_DIM), jnp.int32)

    qt = jnp.swapaxes(qt_ref[...], 0, 1)
    t1 = jnp.sum(qt * jnp.log(qt), axis=0)
    qt_b = qt.astype(jnp.bfloat16).astype(jnp.float32)
    iota_k = lax.broadcasted_iota(jnp.int32, (KB, TEXT_DIM), 0)
    for b in range(B):
        lq_b = lq[b].astype(jnp.bfloat16).astype(jnp.float32)
        t2 = jnp.sum(qt_b * lq_b[:, None, :], axis=0)
        s = t1 - t2
        m = jnp.min(s, axis=0, keepdims=True)
        loc = jnp.min(jnp.where(s == m, iota_k, KB), axis=0, keepdims=True)
        cur_v = minval[b:b + 1, :]
        cur_i = idx_ref[b:b + 1, :]
        better = m < cur_v
        minval[b:b + 1, :] = jnp.where(better, m, cur_v)
        idx_ref[b:b + 1, :] = jnp.where(better, loc + step * KB, cur_i)


def _argmin_call(query, queue_text, interpret=False):
    return pl.pallas_call(
        _argmin_body,
        grid=(NSTEPS,),
        in_specs=[
            pl.BlockSpec((B, WORD_NUM, TEXT_DIM), lambda i: (0, 0, 0)),
            pl.BlockSpec((KB, WORD_NUM, TEXT_DIM), lambda i: (i, 0, 0)),
        ],
        out_specs=pl.BlockSpec((B, TEXT_DIM), lambda i: (0, 0)),
        out_shape=jax.ShapeDtypeStruct((B, TEXT_DIM), jnp.int32),
        scratch_shapes=[
            pltpu.VMEM((B, TEXT_DIM), jnp.float32),
            pltpu.VMEM((B, WORD_NUM, TEXT_DIM), jnp.float32),
        ],
        interpret=interpret,
    )(query, queue_text)


NROWS = B * TEXT_DIM
CH = 4


def _gather_call(idx_flat, table):
    info = plsc.get_sparse_core_info()
    nw = info.num_cores * info.num_subcores
    rows_per_w = NROWS // nw
    nchunk = rows_per_w // CH

    mesh = plsc.VectorSubcoreMesh(core_axis_name="c", subcore_axis_name="s")

    @functools.partial(
        pl.kernel,
        out_type=jax.ShapeDtypeStruct((NROWS, FRAME_NUM, VID_DIM), jnp.float32),
        mesh=mesh,
        scratch_types=[
            pltpu.VMEM((rows_per_w,), jnp.int32),
            pltpu.VMEM((2, CH, FRAME_NUM, VID_DIM), jnp.float32),
            pltpu.SemaphoreType.DMA,
            pltpu.SemaphoreType.DMA,
        ],
    )
    def k(idx_hbm, tab_hbm, out_hbm, idx_v, rows_v, sem_in, sem_out):
        wid = lax.axis_index("s") * info.num_cores + lax.axis_index("c")
        base = wid * rows_per_w
        pltpu.sync_copy(idx_hbm.at[pl.ds(base, rows_per_w)], idx_v)
        out_handles = [None, None]
        for g in range(rows_per_w // 16):
            vec16 = idx_v[pl.ds(g * 16, 16)]
            for cc in range(16 // CH):
                c = g * (16 // CH) + cc
                buf = rows_v.at[c % 2]
                if out_handles[c % 2] is not None:
                    out_handles[c % 2].wait()
                in_handles = []
                for j in range(CH):
                    kidx = vec16[cc * CH + j]
                    in_handles.append(
                        pltpu.async_copy(tab_hbm.at[kidx], buf.at[j], sem_in))
                for h in in_handles:
                    h.wait()
                out_handles[c % 2] = pltpu.async_copy(
                    buf, out_hbm.at[pl.ds(base + c * CH, CH)], sem_out)
        out_handles[0].wait()
        out_handles[1].wait()

    return k(idx_flat, table)


def kernel(query, queue_text, queue_video):
    idx = _argmin_call(query, queue_text)
    out = _gather_call(idx.reshape(NROWS), queue_video)
    return out.reshape(B, TEXT_DIM, FRAME_NUM, VID_DIM)


# speedup vs baseline: 1.0157x; 1.0157x over previous
import functools

import jax
import jax.numpy as jnp
from jax import lax
from jax.experimental import pallas as pl
from jax.experimental.pallas import tpu as pltpu
from jax.experimental.pallas import tpu_sc as plsc

K = 8192
WORD_NUM = 32
TEXT_DIM = 512
FRAME_NUM = 12
VID_DIM = 512
B = 4

KB = 64
NSTEPS = K // KB
ROW = FRAME_NUM * VID_DIM


def _argmin_body(q_ref, qt_ref, idx_ref, minval, lqbc):
    step = pl.program_id(0)

    @pl.when(step == 0)
    def _init():
        minval[...] = jnp.full((B, TEXT_DIM), jnp.inf, jnp.float32)
        idx_ref[...] = jnp.zeros((B, TEXT_DIM), jnp.int32)
        lqbc[...] = jnp.log(q_ref[...]).astype(jnp.bfloat16).astype(jnp.float32)

    qt = jnp.swapaxes(qt_ref[...], 0, 1)
    t1 = jnp.sum(qt * jnp.log(qt), axis=0)
    qt_b = qt.astype(jnp.bfloat16).astype(jnp.float32)
    iota_k = lax.broadcasted_iota(jnp.int32, (KB, TEXT_DIM), 0)
    for b in range(B):
        t2 = jnp.sum(qt_b * lqbc[b][:, None, :], axis=0)
        s = t1 - t2
        m = jnp.min(s, axis=0, keepdims=True)
        loc = jnp.min(jnp.where(s == m, iota_k, KB), axis=0, keepdims=True)
        cur_v = minval[b:b + 1, :]
        cur_i = idx_ref[b:b + 1, :]
        better = m < cur_v
        minval[b:b + 1, :] = jnp.where(better, m, cur_v)
        idx_ref[b:b + 1, :] = jnp.where(better, loc + step * KB, cur_i)


def _argmin_call(query, queue_text, interpret=False):
    return pl.pallas_call(
        _argmin_body,
        grid=(NSTEPS,),
        in_specs=[
            pl.BlockSpec((B, WORD_NUM, TEXT_DIM), lambda i: (0, 0, 0)),
            pl.BlockSpec((KB, WORD_NUM, TEXT_DIM), lambda i: (i, 0, 0)),
        ],
        out_specs=pl.BlockSpec((B, TEXT_DIM), lambda i: (0, 0)),
        out_shape=jax.ShapeDtypeStruct((B, TEXT_DIM), jnp.int32),
        scratch_shapes=[
            pltpu.VMEM((B, TEXT_DIM), jnp.float32),
            pltpu.VMEM((B, WORD_NUM, TEXT_DIM), jnp.float32),
        ],
        interpret=interpret,
    )(query, queue_text)


NROWS = B * TEXT_DIM
CH = 4


def _gather_call(idx_flat, table):
    info = plsc.get_sparse_core_info()
    nw = info.num_cores * info.num_subcores
    rows_per_w = NROWS // nw
    nchunk = rows_per_w // CH

    mesh = plsc.VectorSubcoreMesh(core_axis_name="c", subcore_axis_name="s")

    wper_b = nw // B

    @functools.partial(
        pl.kernel,
        out_type=jax.ShapeDtypeStruct((B, TEXT_DIM, FRAME_NUM, VID_DIM),
                                      jnp.float32),
        mesh=mesh,
        scratch_types=[
            pltpu.VMEM((rows_per_w,), jnp.int32),
            pltpu.VMEM((2, CH, FRAME_NUM, VID_DIM), jnp.float32),
            pltpu.SemaphoreType.DMA,
            pltpu.SemaphoreType.DMA,
        ],
    )
    def k(idx_hbm, tab_hbm, out_hbm, idx_v, rows_v, sem_in, sem_out):
        wid = lax.axis_index("s") * info.num_cores + lax.axis_index("c")
        b_i = wid // wper_b
        dbase = (wid % wper_b) * rows_per_w
        pltpu.sync_copy(idx_hbm.at[b_i, pl.ds(dbase, rows_per_w)], idx_v)
        out_handles = [None, None]
        for g in range(rows_per_w // 16):
            vec16 = idx_v[pl.ds(g * 16, 16)]
            for cc in range(16 // CH):
                c = g * (16 // CH) + cc
                buf = rows_v.at[c % 2]
                if out_handles[c % 2] is not None:
                    out_handles[c % 2].wait()
                in_handles = []
                for j in range(CH):
                    kidx = vec16[cc * CH + j]
                    in_handles.append(
                        pltpu.async_copy(tab_hbm.at[kidx], buf.at[j], sem_in))
                for h in in_handles:
                    h.wait()
                out_handles[c % 2] = pltpu.async_copy(
                    buf, out_hbm.at[b_i, pl.ds(dbase + c * CH, CH)], sem_out)
        out_handles[0].wait()
        out_handles[1].wait()

    return k(idx_flat, table)


def kernel(query, queue_text, queue_video):
    idx = _argmin_call(query, queue_text)
    return _gather_call(idx, queue_video)


# speedup vs baseline: 1.3431x; 1.3223x over previous
import functools

import jax
import jax.numpy as jnp
from jax import lax
from jax.experimental import pallas as pl
from jax.experimental.pallas import tpu as pltpu
from jax.experimental.pallas import tpu_sc as plsc

K = 8192
WORD_NUM = 32
TEXT_DIM = 512
FRAME_NUM = 12
VID_DIM = 512
B = 4

KB = 64
NSTEPS = K // KB
ROW = FRAME_NUM * VID_DIM


def _argmin_body(q_ref, qt_ref, idx_ref, minval, lqbc):
    step = pl.program_id(0)

    @pl.when(step == 0)
    def _init():
        minval[...] = jnp.full((B, TEXT_DIM), jnp.inf, jnp.float32)
        idx_ref[...] = jnp.zeros((B, TEXT_DIM), jnp.int32)
        lqbc[...] = jnp.log(q_ref[...]).astype(jnp.bfloat16).astype(jnp.float32)

    qt = jnp.swapaxes(qt_ref[...], 0, 1)
    t1 = jnp.sum(qt * jnp.log(qt), axis=0)
    qt_b = qt.astype(jnp.bfloat16).astype(jnp.float32)
    iota_k = lax.broadcasted_iota(jnp.int32, (KB, TEXT_DIM), 0)
    for b in range(B):
        t2 = jnp.sum(qt_b * lqbc[b][:, None, :], axis=0)
        s = t1 - t2
        m = jnp.min(s, axis=0, keepdims=True)
        loc = jnp.min(jnp.where(s == m, iota_k, KB), axis=0, keepdims=True)
        cur_v = minval[b:b + 1, :]
        cur_i = idx_ref[b:b + 1, :]
        better = m < cur_v
        minval[b:b + 1, :] = jnp.where(better, m, cur_v)
        idx_ref[b:b + 1, :] = jnp.where(better, loc + step * KB, cur_i)


def _argmin_call(query, queue_text, interpret=False):
    return pl.pallas_call(
        _argmin_body,
        grid=(NSTEPS,),
        in_specs=[
            pl.BlockSpec((B, WORD_NUM, TEXT_DIM), lambda i: (0, 0, 0)),
            pl.BlockSpec((KB, WORD_NUM, TEXT_DIM), lambda i: (i, 0, 0)),
        ],
        out_specs=pl.BlockSpec((B, TEXT_DIM), lambda i: (0, 0)),
        out_shape=jax.ShapeDtypeStruct((B, TEXT_DIM), jnp.int32),
        scratch_shapes=[
            pltpu.VMEM((B, TEXT_DIM), jnp.float32),
            pltpu.VMEM((B, WORD_NUM, TEXT_DIM), jnp.float32),
        ],
        interpret=interpret,
    )(query, queue_text)


NROWS = B * TEXT_DIM
CH = 16


def _gather_call(idx, table_f):
    info = plsc.get_sparse_core_info()
    nw = info.num_cores * info.num_subcores
    rows_per_w = NROWS // nw
    nchunk = rows_per_w // CH
    wper_b = nw // B

    mesh = plsc.VectorSubcoreMesh(core_axis_name="c", subcore_axis_name="s")

    @functools.partial(
        pl.kernel,
        out_type=jax.ShapeDtypeStruct((B, FRAME_NUM, TEXT_DIM, VID_DIM),
                                      jnp.float32),
        mesh=mesh,
        scratch_types=[
            pltpu.VMEM((rows_per_w,), jnp.int32),
            pltpu.VMEM((FRAME_NUM, CH, VID_DIM), jnp.float32),
            pltpu.SemaphoreType.DMA,
            pltpu.SemaphoreType.DMA,
        ],
    )
    def k(idx_hbm, tab_hbm, out_hbm, idx_v, fbuf, sem_in, sem_out):
        wid = lax.axis_index("s") * info.num_cores + lax.axis_index("c")
        b_i = wid // wper_b
        dbase = (wid % wper_b) * rows_per_w
        pltpu.sync_copy(idx_hbm.at[b_i, pl.ds(dbase, rows_per_w)], idx_v)

        def chunk(c, carry):
            vec16 = idx_v[pl.ds(c * CH, CH)]
            in_handles = []
            for f in range(FRAME_NUM):
                for j in range(CH):
                    in_handles.append(pltpu.async_copy(
                        tab_hbm.at[f, vec16[j]], fbuf.at[f, j], sem_in))
            for h in in_handles:
                h.wait()
            out_handles = []
            for f in range(FRAME_NUM):
                out_handles.append(pltpu.async_copy(
                    fbuf.at[f],
                    out_hbm.at[b_i, f, pl.ds(dbase + c * CH, CH)], sem_out))
            for h in out_handles:
                h.wait()
            return carry

        lax.fori_loop(0, nchunk, chunk, 0)

    return k(idx, table_f)


def kernel(query, queue_text, queue_video):
    idx = _argmin_call(query, queue_text)
    table_f = jnp.transpose(queue_video, (1, 0, 2))
    out_f = _gather_call(idx, table_f)
    return jnp.transpose(out_f, (0, 2, 1, 3))
